# baseline (speedup 1.0000x reference)
import functools

import jax
import jax.numpy as jnp
from jax import lax
from jax.experimental import pallas as pl
from jax.experimental.pallas import tpu as pltpu
from jax.experimental.pallas import tpu_sc as plsc

NU = 10000
HD = 128
EK = 64


def _span_chunks(spp):
    ch = (spp + EK - 1) // EK
    return ch + (ch % 2)
CNTW = 16
BLK = 1000


def _make_sc_agg(n_rel, e_total, with_cnt):
    info = plsc.get_sparse_core_info()
    NC, NS = info.num_cores, info.num_subcores
    NW = NC * NS
    spp = e_total // NW
    CH = _span_chunks(spp)
    nzc = NU // EK + 1
    TAIL = NU - EK
    kmax = (nzc + NS - 1) // NS

    out_type = [jax.ShapeDtypeStruct((NC * n_rel * NU, HD), jnp.float32)]
    scratch = [
        pltpu.VMEM((EK,), jnp.int32),
        pltpu.VMEM((EK,), jnp.int32),
        pltpu.VMEM((EK,), jnp.int32),
        pltpu.VMEM((EK,), jnp.int32),
        pltpu.VMEM((EK,), jnp.int32),
        pltpu.VMEM((EK, HD), jnp.float32),
        pltpu.VMEM((EK, HD), jnp.float32),
        pltpu.SemaphoreType.DMA,
        pltpu.SemaphoreType.DMA,
        pltpu.VMEM_SHARED((NU + 8, HD), jnp.float32),
    ]
    if with_cnt:
        out_type.append(jax.ShapeDtypeStruct((NC * n_rel * NU, HD), jnp.float32))

    mesh = plsc.VectorSubcoreMesh(core_axis_name="c", subcore_axis_name="s")

    @functools.partial(pl.kernel, mesh=mesh, out_type=out_type,
                       scratch_types=scratch)
    def sc_agg(x_hbm, src_hbm, dst_hbm, zg_hbm, iota_hbm, ones_hbm, *rest):
        if with_cnt:
            (g_out, cnt_out, srcA, srcB, dstA, dstB, idn_v,
             rowsA, rowsB, semA, semB, shared_g) = rest
        else:
            (g_out, srcA, srcB, dstA, dstB, idn_v,
             rowsA, rowsB, semA, semB, shared_g) = rest
        c = lax.axis_index("c")
        s = lax.axis_index("s")
        w = s * NC + c

        phases = [0, 1] if with_cnt else [0]
        for r in range(n_rel):
            for phase in phases:
                pltpu.sync_copy(zg_hbm.at[pl.ds(0, EK)], rowsA)

                def zero_chunk(k, _):
                    zc = k * NS + s
                    @pl.when(zc < nzc)
                    def _():
                        row = pl.multiple_of(jnp.minimum(zc * EK, TAIL), 8)
                        pltpu.sync_copy(iota_hbm.at[pl.ds(row, EK)], idn_v)
                        pltpu.sync_copy(rowsA, shared_g.at[idn_v])
                    return 0
                lax.fori_loop(0, kmax, zero_chunk, 0)
                plsc.subcore_barrier()

                span0 = ((r * NW + w) * CH) * EK

                if phase == 0:
                    base0 = pl.multiple_of(span0, 8)
                    pltpu.sync_copy(src_hbm.at[pl.ds(base0, EK)], srcA)
                    pltpu.sync_copy(dst_hbm.at[pl.ds(base0, EK)], dstA)
                    dA = pltpu.async_copy(x_hbm.at[srcA], rowsA, semA)

                    def pair(jj, _):
                        j0 = jj * 2
                        baseB = pl.multiple_of(span0 + (j0 + 1) * EK, 8)
                        pltpu.sync_copy(src_hbm.at[pl.ds(baseB, EK)], srcB)
                        pltpu.sync_copy(dst_hbm.at[pl.ds(baseB, EK)], dstB)
                        dB = pltpu.async_copy(x_hbm.at[srcB], rowsB, semB)
                        dA2 = pltpu.make_async_copy(x_hbm.at[srcA], rowsA, semA)
                        dA2.wait()
                        pltpu.sync_copy(rowsA, shared_g.at[dstA], add=True)
                        @pl.when(jj < CH // 2 - 1)
                        def _():
                            baseA = pl.multiple_of(span0 + (j0 + 2) * EK, 8)
                            pltpu.sync_copy(src_hbm.at[pl.ds(baseA, EK)], srcA)
                            pltpu.sync_copy(dst_hbm.at[pl.ds(baseA, EK)], dstA)
                            pltpu.async_copy(x_hbm.at[srcA], rowsA, semA)
                        dB.wait()
                        pltpu.sync_copy(rowsB, shared_g.at[dstB], add=True)
                        return 0
                    lax.fori_loop(0, CH // 2, pair, 0)
                else:
                    pltpu.sync_copy(ones_hbm.at[pl.ds(0, EK)], rowsA)

                    def cchunk(j, _):
                        base = pl.multiple_of(span0 + j * EK, 8)
                        pltpu.sync_copy(dst_hbm.at[pl.ds(base, EK)], dstA)
                        pltpu.sync_copy(rowsA, shared_g.at[dstA], add=True)
                        return 0
                    lax.fori_loop(0, CH, cchunk, 0)
                plsc.subcore_barrier()

                out_base = (c * n_rel + r) * NU
                out_ref = g_out if phase == 0 else cnt_out
                def write_chunk(k, _):
                    zc = k * NS + s
                    @pl.when(zc < nzc)
                    def _():
                        row = pl.multiple_of(jnp.minimum(zc * EK, TAIL), 8)
                        orow = pl.multiple_of(
                            out_base + jnp.minimum(zc * EK, TAIL), 8)
                        pltpu.sync_copy(iota_hbm.at[pl.ds(row, EK)], idn_v)
                        pltpu.async_copy(shared_g.at[idn_v], rowsA, semA).wait()
                        pltpu.sync_copy(rowsA, out_ref.at[pl.ds(orow, EK)])
                    return 0
                lax.fori_loop(0, kmax, write_chunk, 0)
                plsc.subcore_barrier()

    return sc_agg


def _proj_body(x_ref, w_ref, b_ref, o_ref):
    o_ref[...] = (jnp.dot(x_ref[...], w_ref[...],
                          preferred_element_type=jnp.float32) + b_ref[...])


def _inv(ca, cb):
    return 1.0 / jnp.maximum(ca[:, :1] + cb[:, :1], 1.0)


def _layer_user_body(x_ref, giuA, giuB, guuA, guuB, ciuA, ciuB, cuuA, cuuB,
                     wroot, w1, w2, b_ref, o_ref):
    acc = jnp.dot(x_ref[...], wroot[...], preferred_element_type=jnp.float32)
    giu = (giuA[...] + giuB[...]) * _inv(ciuA, ciuB)
    acc += jnp.dot(giu, w1[...], preferred_element_type=jnp.float32)
    guu = (guuA[...] + guuB[...]) * _inv(cuuA, cuuB)
    acc += jnp.dot(guu, w2[...], preferred_element_type=jnp.float32)
    o_ref[...] = jnp.maximum(acc + b_ref[...], 0.0)


def _layer_item_body(x_ref, gucA, gucB, cucA, cucB, wroot, w0, b_ref, o_ref):
    acc = jnp.dot(x_ref[...], wroot[...], preferred_element_type=jnp.float32)
    guc = (gucA[...] + gucB[...]) * _inv(cucA, cucB)
    acc += jnp.dot(guc, w0[...], preferred_element_type=jnp.float32)
    o_ref[...] = jnp.maximum(acc + b_ref[...], 0.0)


def _final_user_body(x_ref, giuA, giuB, guuA, guuB, ciuA, ciuB, cuuA, cuuB,
                     wroot, w1, w2, b_ref, wout, bout, o_ref):
    acc = jnp.dot(x_ref[...], wroot[...], preferred_element_type=jnp.float32)
    giu = (giuA[...] + giuB[...]) * _inv(ciuA, ciuB)
    acc += jnp.dot(giu, w1[...], preferred_element_type=jnp.float32)
    guu = (guuA[...] + guuB[...]) * _inv(cuuA, cuuB)
    acc += jnp.dot(guu, w2[...], preferred_element_type=jnp.float32)
    h = jnp.maximum(acc + b_ref[...], 0.0)
    o_ref[...] = jnp.dot(h, wout[...], preferred_element_type=jnp.float32) + bout[...]


def _g_spec(core, rel):
    return pl.BlockSpec((None, None, BLK, HD), lambda j, c=core, r=rel: (c, r, j, 0))


def _c_spec(core, rel):
    return pl.BlockSpec((None, None, BLK, CNTW), lambda j, c=core, r=rel: (c, r, j, 0))


def _w_spec():
    return pl.BlockSpec((HD, HD), lambda j: (0, 0))


def _b_spec(width=HD):
    return pl.BlockSpec((1, width), lambda j: (0, 0))


def kernel(x_user, x_item, edge_index_uc, edge_index_iu, edge_index_uu,
           lin_user_W, lin_user_b, lin_item_W, lin_item_b,
           W_rel, W_root, conv_b, out_W, out_b):
    n_user, d = x_user.shape
    n_item = x_item.shape[0]
    nn = n_user + n_item
    e = edge_index_uc.shape[1]
    h = W_root.shape[-1]
    cdim = out_W.shape[1]
    nub = n_user // BLK

    ei_uc = edge_index_uc.astype(jnp.int32)
    ei_iu = edge_index_iu.astype(jnp.int32)
    ei_uu = edge_index_uu.astype(jnp.int32)
    info = plsc.get_sparse_core_info()
    nw = info.num_cores * info.num_subcores
    spp = e // nw
    ch = _span_chunks(spp)

    def _span_pack(flat, fill):
        a = flat.reshape(-1, nw, spp)
        a = jnp.pad(a, ((0, 0), (0, 0), (0, ch * EK - spp)),
                    constant_values=fill)
        return a.reshape(-1)

    src1 = _span_pack(jnp.stack([ei_uc[0], ei_iu[0] + n_user, ei_uu[0]]), 0)
    dst1 = _span_pack(jnp.stack([ei_uc[1], ei_iu[1], ei_uu[1]]), NU)
    src2 = src1[nw * ch * EK:]
    dst2 = dst1[nw * ch * EK:]

    xs = jnp.stack([x_user, x_item])
    ws = jnp.stack([lin_user_W, lin_item_W])
    bs = jnp.stack([lin_user_b, lin_item_b]).reshape(2, 1, h)
    x0 = pl.pallas_call(
        _proj_body,
        grid=(2, nub),
        in_specs=[
            pl.BlockSpec((None, BLK, d), lambda i, j: (i, j, 0)),
            pl.BlockSpec((None, d, h), lambda i, j: (i, 0, 0)),
            pl.BlockSpec((None, 1, h), lambda i, j: (i, 0, 0)),
        ],
        out_specs=pl.BlockSpec((BLK, h), lambda i, j: (i * nub + j, 0)),
        out_shape=jax.ShapeDtypeStruct((nn, h), jnp.float32),
    )(xs, ws, bs)

    zg = jnp.zeros((NU, h), jnp.float32)
    iota = jnp.arange(NU, dtype=jnp.int32)
    ones128 = jnp.ones((EK, h), jnp.float32)
    g1, cntw = _make_sc_agg(3, e, True)(x0, src1, dst1, zg, iota, ones128)
    g1 = g1.reshape(2, 3, NU, h)
    cnt = cntw.reshape(2, 3, NU, h)[..., :CNTW]

    user_in_specs = [
        pl.BlockSpec((BLK, h), lambda j: (j, 0)),
        _g_spec(0, 1), _g_spec(1, 1), _g_spec(0, 2), _g_spec(1, 2),
        _c_spec(0, 1), _c_spec(1, 1), _c_spec(0, 2), _c_spec(1, 2),
        _w_spec(), _w_spec(), _w_spec(), _b_spec(),
    ]
    item_in_specs = [
        pl.BlockSpec((BLK, h), lambda j: (j + NU // BLK, 0)),
        _g_spec(0, 0), _g_spec(1, 0),
        _c_spec(0, 0), _c_spec(1, 0),
        _w_spec(), _w_spec(), _b_spec(),
    ]
    cb0 = conv_b[0].reshape(1, h)
    x1u = pl.pallas_call(
        _layer_user_body, grid=(nub,),
        in_specs=user_in_specs,
        out_specs=pl.BlockSpec((BLK, h), lambda j: (j, 0)),
        out_shape=jax.ShapeDtypeStruct((n_user, h), jnp.float32),
    )(x0, g1, g1, g1, g1, cnt, cnt, cnt, cnt,
      W_root[0], W_rel[0, 1], W_rel[0, 2], cb0)
    x1i = pl.pallas_call(
        _layer_item_body, grid=(nub,),
        in_specs=item_in_specs,
        out_specs=pl.BlockSpec((BLK, h), lambda j: (j, 0)),
        out_shape=jax.ShapeDtypeStruct((n_item, h), jnp.float32),
    )(x0, g1, g1, cnt, cnt, W_root[0], W_rel[0, 0], cb0)
    x1 = jnp.concatenate([x1u, x1i], axis=0)

    (g2,) = _make_sc_agg(2, e, False)(x1, src2, dst2, zg, iota, ones128)
    g2 = g2.reshape(2, 2, NU, h)

    final_in_specs = [
        pl.BlockSpec((BLK, h), lambda j: (j, 0)),
        _g_spec(0, 0), _g_spec(1, 0), _g_spec(0, 1), _g_spec(1, 1),
        _c_spec(0, 1), _c_spec(1, 1), _c_spec(0, 2), _c_spec(1, 2),
        _w_spec(), _w_spec(), _w_spec(), _b_spec(),
        pl.BlockSpec((h, cdim), lambda j: (0, 0)), _b_spec(cdim),
    ]
    out = pl.pallas_call(
        _final_user_body, grid=(nub,),
        in_specs=final_in_specs,
        out_specs=pl.BlockSpec((BLK, cdim), lambda j: (j, 0)),
        out_shape=jax.ShapeDtypeStruct((n_user, cdim), jnp.float32),
    )(x1, g2, g2, g2, g2, cnt, cnt, cnt, cnt,
      W_root[1], W_rel[1, 1], W_rel[1, 2], conv_b[1].reshape(1, h),
      out_W, out_b.reshape(1, cdim))
    return out


# speedup vs baseline: 1.7039x; 1.7039x over previous
import functools

import jax
import jax.numpy as jnp
from jax import lax
from jax.experimental import pallas as pl
from jax.experimental.pallas import tpu as pltpu
from jax.experimental.pallas import tpu_sc as plsc

NU = 10000
HD = 128
EK = 128
CNTW = 16
BLK = 1000


def _make_sc_agg(n_rel, e_total, with_cnt):
    info = plsc.get_sparse_core_info()
    NC, NS = info.num_cores, info.num_subcores
    NW = NC * NS
    assert e_total % EK == 0
    nch = e_total // EK
    jmax = (nch + NW - 1) // NW
    nzc = NU // EK + 1
    TAIL = NU - EK
    kmax = (nzc + NS - 1) // NS

    out_type = [jax.ShapeDtypeStruct((NC * n_rel * NU, HD), jnp.float32)]
    scratch = [
        pltpu.VMEM((EK,), jnp.int32),
        pltpu.VMEM((EK,), jnp.int32),
        pltpu.VMEM((EK,), jnp.int32),
        pltpu.VMEM((EK, HD), jnp.float32),
        pltpu.SemaphoreType.DMA,
        pltpu.VMEM_SHARED((NU, HD), jnp.float32),
    ]
    if with_cnt:
        out_type.append(jax.ShapeDtypeStruct((NC * n_rel * NU, HD), jnp.float32))

    mesh = plsc.VectorSubcoreMesh(core_axis_name="c", subcore_axis_name="s")

    @functools.partial(pl.kernel, mesh=mesh, out_type=out_type,
                       scratch_types=scratch)
    def sc_agg(x_hbm, src_hbm, dst_hbm, zg_hbm, iota_hbm, ones_hbm, *rest):
        if with_cnt:
            g_out, cnt_out, src_v, dst_v, idn_v, rows_v, sem, shared_g = rest
        else:
            g_out, src_v, dst_v, idn_v, rows_v, sem, shared_g = rest
        c = lax.axis_index("c")
        s = lax.axis_index("s")
        w = s * NC + c

        phases = [0, 1] if with_cnt else [0]
        for r in range(n_rel):
            for phase in phases:
                pltpu.sync_copy(zg_hbm.at[pl.ds(0, EK)], rows_v)

                def zero_chunk(k, _):
                    zc = k * NS + s
                    @pl.when(zc < nzc)
                    def _():
                        row = pl.multiple_of(jnp.minimum(zc * EK, TAIL), 8)
                        pltpu.sync_copy(iota_hbm.at[pl.ds(row, EK)], idn_v)
                        pltpu.sync_copy(rows_v, shared_g.at[idn_v])
                    return 0
                lax.fori_loop(0, kmax, zero_chunk, 0)
                if phase == 1:
                    pltpu.sync_copy(ones_hbm.at[pl.ds(0, EK)], rows_v)
                plsc.subcore_barrier()

                def chunk(j, _):
                    ch = j * NW + w
                    @pl.when(ch < nch)
                    def _():
                        base = pl.multiple_of(r * e_total + ch * EK, 8)
                        pltpu.sync_copy(dst_hbm.at[pl.ds(base, EK)], dst_v)
                        if phase == 0:
                            pltpu.sync_copy(src_hbm.at[pl.ds(base, EK)], src_v)
                            pltpu.async_copy(x_hbm.at[src_v], rows_v, sem).wait()
                        pltpu.sync_copy(rows_v, shared_g.at[dst_v], add=True)
                    return 0
                lax.fori_loop(0, jmax, chunk, 0)
                plsc.subcore_barrier()

                out_base = (c * n_rel + r) * NU
                out_ref = g_out if phase == 0 else cnt_out
                def write_chunk(k, _):
                    zc = k * NS + s
                    @pl.when(zc < nzc)
                    def _():
                        row = pl.multiple_of(jnp.minimum(zc * EK, TAIL), 8)
                        orow = pl.multiple_of(
                            out_base + jnp.minimum(zc * EK, TAIL), 8)
                        pltpu.sync_copy(iota_hbm.at[pl.ds(row, EK)], idn_v)
                        pltpu.async_copy(shared_g.at[idn_v], rows_v, sem).wait()
                        pltpu.sync_copy(rows_v, out_ref.at[pl.ds(orow, EK)])
                    return 0
                lax.fori_loop(0, kmax, write_chunk, 0)
                plsc.subcore_barrier()

    return sc_agg


def _proj_body(x_ref, w_ref, b_ref, o_ref):
    o_ref[...] = (jnp.dot(x_ref[...], w_ref[...],
                          preferred_element_type=jnp.float32) + b_ref[...])


def _inv(ca, cb):
    return 1.0 / jnp.maximum(ca[:, :1] + cb[:, :1], 1.0)


def _layer_user_body(x_ref, giuA, giuB, guuA, guuB, ciuA, ciuB, cuuA, cuuB,
                     wroot, w1, w2, b_ref, o_ref):
    acc = jnp.dot(x_ref[...], wroot[...], preferred_element_type=jnp.float32)
    giu = (giuA[...] + giuB[...]) * _inv(ciuA, ciuB)
    acc += jnp.dot(giu, w1[...], preferred_element_type=jnp.float32)
    guu = (guuA[...] + guuB[...]) * _inv(cuuA, cuuB)
    acc += jnp.dot(guu, w2[...], preferred_element_type=jnp.float32)
    o_ref[...] = jnp.maximum(acc + b_ref[...], 0.0)


def _layer_item_body(x_ref, gucA, gucB, cucA, cucB, wroot, w0, b_ref, o_ref):
    acc = jnp.dot(x_ref[...], wroot[...], preferred_element_type=jnp.float32)
    guc = (gucA[...] + gucB[...]) * _inv(cucA, cucB)
    acc += jnp.dot(guc, w0[...], preferred_element_type=jnp.float32)
    o_ref[...] = jnp.maximum(acc + b_ref[...], 0.0)


def _final_user_body(x_ref, giuA, giuB, guuA, guuB, ciuA, ciuB, cuuA, cuuB,
                     wroot, w1, w2, b_ref, wout, bout, o_ref):
    acc = jnp.dot(x_ref[...], wroot[...], preferred_element_type=jnp.float32)
    giu = (giuA[...] + giuB[...]) * _inv(ciuA, ciuB)
    acc += jnp.dot(giu, w1[...], preferred_element_type=jnp.float32)
    guu = (guuA[...] + guuB[...]) * _inv(cuuA, cuuB)
    acc += jnp.dot(guu, w2[...], preferred_element_type=jnp.float32)
    h = jnp.maximum(acc + b_ref[...], 0.0)
    o_ref[...] = jnp.dot(h, wout[...], preferred_element_type=jnp.float32) + bout[...]


def _g_spec(core, rel):
    return pl.BlockSpec((None, None, BLK, HD), lambda j, c=core, r=rel: (c, r, j, 0))


def _c_spec(core, rel):
    return pl.BlockSpec((None, None, BLK, CNTW), lambda j, c=core, r=rel: (c, r, j, 0))


def _w_spec():
    return pl.BlockSpec((HD, HD), lambda j: (0, 0))


def _b_spec(width=HD):
    return pl.BlockSpec((1, width), lambda j: (0, 0))


def kernel(x_user, x_item, edge_index_uc, edge_index_iu, edge_index_uu,
           lin_user_W, lin_user_b, lin_item_W, lin_item_b,
           W_rel, W_root, conv_b, out_W, out_b):
    n_user, d = x_user.shape
    n_item = x_item.shape[0]
    nn = n_user + n_item
    e = edge_index_uc.shape[1]
    h = W_root.shape[-1]
    cdim = out_W.shape[1]
    nub = n_user // BLK

    ei_uc = edge_index_uc.astype(jnp.int32)
    ei_iu = edge_index_iu.astype(jnp.int32)
    ei_uu = edge_index_uu.astype(jnp.int32)
    src1 = jnp.stack([ei_uc[0], ei_iu[0] + n_user, ei_uu[0]]).reshape(-1)
    dst1 = jnp.stack([ei_uc[1], ei_iu[1], ei_uu[1]]).reshape(-1)
    src2 = src1[e:]
    dst2 = dst1[e:]

    xs = jnp.stack([x_user, x_item])
    ws = jnp.stack([lin_user_W, lin_item_W])
    bs = jnp.stack([lin_user_b, lin_item_b]).reshape(2, 1, h)
    x0 = pl.pallas_call(
        _proj_body,
        grid=(2, nub),
        in_specs=[
            pl.BlockSpec((None, BLK, d), lambda i, j: (i, j, 0)),
            pl.BlockSpec((None, d, h), lambda i, j: (i, 0, 0)),
            pl.BlockSpec((None, 1, h), lambda i, j: (i, 0, 0)),
        ],
        out_specs=pl.BlockSpec((BLK, h), lambda i, j: (i * nub + j, 0)),
        out_shape=jax.ShapeDtypeStruct((nn, h), jnp.float32),
    )(xs, ws, bs)

    zg = jnp.zeros((NU, h), jnp.float32)
    iota = jnp.arange(NU, dtype=jnp.int32)
    ones128 = jnp.ones((EK, h), jnp.float32)
    g1, cntw = _make_sc_agg(3, e, True)(x0, src1, dst1, zg, iota, ones128)
    g1 = g1.reshape(2, 3, NU, h)
    cnt = cntw.reshape(2, 3, NU, h)[..., :CNTW]

    user_in_specs = [
        pl.BlockSpec((BLK, h), lambda j: (j, 0)),
        _g_spec(0, 1), _g_spec(1, 1), _g_spec(0, 2), _g_spec(1, 2),
        _c_spec(0, 1), _c_spec(1, 1), _c_spec(0, 2), _c_spec(1, 2),
        _w_spec(), _w_spec(), _w_spec(), _b_spec(),
    ]
    item_in_specs = [
        pl.BlockSpec((BLK, h), lambda j: (j + NU // BLK, 0)),
        _g_spec(0, 0), _g_spec(1, 0),
        _c_spec(0, 0), _c_spec(1, 0),
        _w_spec(), _w_spec(), _b_spec(),
    ]
    cb0 = conv_b[0].reshape(1, h)
    x1u = pl.pallas_call(
        _layer_user_body, grid=(nub,),
        in_specs=user_in_specs,
        out_specs=pl.BlockSpec((BLK, h), lambda j: (j, 0)),
        out_shape=jax.ShapeDtypeStruct((n_user, h), jnp.float32),
    )(x0, g1, g1, g1, g1, cnt, cnt, cnt, cnt,
      W_root[0], W_rel[0, 1], W_rel[0, 2], cb0)
    x1i = pl.pallas_call(
        _layer_item_body, grid=(nub,),
        in_specs=item_in_specs,
        out_specs=pl.BlockSpec((BLK, h), lambda j: (j, 0)),
        out_shape=jax.ShapeDtypeStruct((n_item, h), jnp.float32),
    )(x0, g1, g1, cnt, cnt, W_root[0], W_rel[0, 0], cb0)
    x1 = jnp.concatenate([x1u, x1i], axis=0)

    (g2,) = _make_sc_agg(2, e, False)(x1, src2, dst2, zg, iota, ones128)
    g2 = g2.reshape(2, 2, NU, h)

    final_in_specs = [
        pl.BlockSpec((BLK, h), lambda j: (j, 0)),
        _g_spec(0, 0), _g_spec(1, 0), _g_spec(0, 1), _g_spec(1, 1),
        _c_spec(0, 1), _c_spec(1, 1), _c_spec(0, 2), _c_spec(1, 2),
        _w_spec(), _w_spec(), _w_spec(), _b_spec(),
        pl.BlockSpec((h, cdim), lambda j: (0, 0)), _b_spec(cdim),
    ]
    out = pl.pallas_call(
        _final_user_body, grid=(nub,),
        in_specs=final_in_specs,
        out_specs=pl.BlockSpec((BLK, cdim), lambda j: (j, 0)),
        out_shape=jax.ShapeDtypeStruct((n_user, cdim), jnp.float32),
    )(x1, g2, g2, g2, g2, cnt, cnt, cnt, cnt,
      W_root[1], W_rel[1, 1], W_rel[1, 2], conv_b[1].reshape(1, h),
      out_W, out_b.reshape(1, cdim))
    return out
